# baseline (speedup 1.0000x reference)
import functools

import jax
import jax.numpy as jnp
from jax import lax
from jax.experimental import pallas as pl
from jax.experimental.pallas import tpu as pltpu
from jax.experimental.pallas import tpu_sc as plsc

BATCH = 4096
HIST = 50
D = 64
VOCAB = 100000

NC = 2
NS = 16
NW = NC * NS
DPW = D // NW
LANES = 16
NVEC = BATCH // LANES
UNROLL = 32
LSTEP = 2
NL = HIST // LSTEP

_mesh = plsc.VectorSubcoreMesh(core_axis_name="c", subcore_axis_name="s")


@functools.partial(
    pl.kernel,
    mesh=_mesh,
    out_type=jax.ShapeDtypeStruct((HIST, D // 8, BATCH // 128, 8, 128),
                                  jnp.float32),
    scratch_types=[
        pltpu.VMEM((VOCAB,), jnp.float32),
        pltpu.VMEM((2, LSTEP, BATCH), jnp.int32),
        pltpu.VMEM((LSTEP, BATCH // 128, 128), jnp.float32),
        pltpu.SemaphoreType.DMA,
        pltpu.SemaphoreType.DMA,
    ],
    compiler_params=pltpu.CompilerParams(
        use_tc_tiling_on_sc=True, needs_layout_passes=False
    ),
)
def _emb_tgather(tab_hbm, idx_hbm, out_hbm, row_v, idx_v, res_v,
                 sem_i, sem_o):
    wid = lax.axis_index("s") * NC + lax.axis_index("c")

    for d_i in range(DPW):
        d = wid * DPW + d_i
        d_hi = d // 8
        d_lo = d % 8
        pltpu.sync_copy(tab_hbm.at[d], row_v)
        pltpu.async_copy(idx_hbm.at[pl.ds(0, LSTEP)], idx_v.at[0], sem_i).wait()

        def body(k, carry):
            bi = k % 2
            @pl.when(k < NL - 1)
            def _pre():
                pltpu.make_async_copy(
                    idx_hbm.at[pl.ds((k + 1) * LSTEP, LSTEP)],
                    idx_v.at[1 - bi],
                    sem_i,
                ).start()

            for j in range(LSTEP):
                @pl.when(k >= 1)
                def _drain():
                    pltpu.make_async_copy(
                        res_v.at[j], out_hbm.at[0, d_hi, :, d_lo, :], sem_o
                    ).wait()

                @plsc.parallel_loop(0, NVEC, unroll=UNROLL)
                def _gather(c):
                    iv = idx_v[bi, j, pl.ds(c * LANES, LANES)]
                    g = plsc.load_gather(row_v, (iv,))
                    res_v[j, c >> 3, pl.ds((c & 7) * LANES, LANES)] = g

                pltpu.make_async_copy(
                    res_v.at[j],
                    out_hbm.at[k * LSTEP + j, d_hi, :, d_lo, :],
                    sem_o,
                ).start()

            @pl.when(k < NL - 1)
            def _wait_pre():
                pltpu.make_async_copy(
                    idx_hbm.at[pl.ds((k + 1) * LSTEP, LSTEP)],
                    idx_v.at[1 - bi],
                    sem_i,
                ).wait()

            return carry

        lax.fori_loop(0, NL, body, 0)

        for j in range(LSTEP):
            pltpu.make_async_copy(
                res_v.at[j], out_hbm.at[0, d_hi, :, d_lo, :], sem_o
            ).wait()


def kernel(indices, table):
    idx_t = indices.astype(jnp.int32).T
    tab_t = table.T
    out5 = _emb_tgather(tab_t, idx_t)
    return out5.transpose(2, 4, 0, 1, 3).reshape(BATCH, HIST, D)


# speedup vs baseline: 1.0129x; 1.0129x over previous
import functools

import jax
import jax.numpy as jnp
from jax import lax
from jax.experimental import pallas as pl
from jax.experimental.pallas import tpu as pltpu
from jax.experimental.pallas import tpu_sc as plsc

BATCH = 4096
HIST = 50
D = 64
VOCAB = 100000

NC = 2
NS = 16
NW = NC * NS
DPW = D // NW
LANES = 16
NVEC = BATCH // LANES
UNROLL = 16
LSTEP = 2
NL = HIST // LSTEP

_mesh = plsc.VectorSubcoreMesh(core_axis_name="c", subcore_axis_name="s")


@functools.partial(
    pl.kernel,
    mesh=_mesh,
    out_type=jax.ShapeDtypeStruct((HIST, D // 8, BATCH // 128, 8, 128),
                                  jnp.float32),
    scratch_types=[
        pltpu.VMEM((VOCAB,), jnp.float32),
        pltpu.VMEM((2, LSTEP, BATCH), jnp.int32),
        pltpu.VMEM((LSTEP, BATCH // 128, 128), jnp.float32),
        pltpu.SemaphoreType.DMA,
        pltpu.SemaphoreType.DMA,
    ],
    compiler_params=pltpu.CompilerParams(
        use_tc_tiling_on_sc=True, needs_layout_passes=False
    ),
)
def _emb_tgather(tab_hbm, idx_hbm, out_hbm, row_v, idx_v, res_v,
                 sem_i, sem_o):
    wid = lax.axis_index("s") * NC + lax.axis_index("c")

    for d_i in range(DPW):
        d = wid * DPW + d_i
        d_hi = d // 8
        d_lo = d % 8
        pltpu.sync_copy(tab_hbm.at[d], row_v)
        pltpu.async_copy(idx_hbm.at[pl.ds(0, LSTEP)], idx_v.at[0], sem_i).wait()

        def body(k, carry):
            bi = k % 2
            @pl.when(k < NL - 1)
            def _pre():
                pltpu.make_async_copy(
                    idx_hbm.at[pl.ds((k + 1) * LSTEP, LSTEP)],
                    idx_v.at[1 - bi],
                    sem_i,
                ).start()

            for j in range(LSTEP):
                @pl.when(k >= 1)
                def _drain():
                    pltpu.make_async_copy(
                        res_v.at[j], out_hbm.at[0, d_hi, :, d_lo, :], sem_o
                    ).wait()

                @plsc.parallel_loop(0, NVEC, unroll=UNROLL)
                def _gather(c):
                    iv = idx_v[bi, j, pl.ds(c * LANES, LANES)]
                    g = plsc.load_gather(row_v, (iv,))
                    res_v[j, c >> 3, pl.ds((c & 7) * LANES, LANES)] = g

                pltpu.make_async_copy(
                    res_v.at[j],
                    out_hbm.at[k * LSTEP + j, d_hi, :, d_lo, :],
                    sem_o,
                ).start()

            @pl.when(k < NL - 1)
            def _wait_pre():
                pltpu.make_async_copy(
                    idx_hbm.at[pl.ds((k + 1) * LSTEP, LSTEP)],
                    idx_v.at[1 - bi],
                    sem_i,
                ).wait()

            return carry

        lax.fori_loop(0, NL, body, 0)

        for j in range(LSTEP):
            pltpu.make_async_copy(
                res_v.at[j], out_hbm.at[0, d_hi, :, d_lo, :], sem_o
            ).wait()


def kernel(indices, table):
    idx_t = indices.astype(jnp.int32).T
    tab_t = table.T
    out5 = _emb_tgather(tab_t, idx_t)
    return out5.transpose(2, 4, 0, 1, 3).reshape(BATCH, HIST, D)
